# Initial kernel scaffold; baseline (speedup 1.0000x reference)
import functools

import jax
import jax.numpy as jnp
from jax import lax
from jax.experimental import pallas as pl
from jax.experimental.pallas import tpu as pltpu
from jax.experimental.pallas import tpu_sc as plsc

_VOCAB = 100000
_E = 8
_D_SUB = 128
_D_TRAIN = 512
_B, _L = 1024, 50
_N = _B * _L
_NC = 2
_NS = 16
_NW = _NC * _NS
_TPW = _N // _NW
_CT = 80
_NCH = _TPW // _CT


def _body(x_hbm, t8_hbm, w4_hbm, out_hbm, xv, idx8, idxb, buf_a, buf_b,
          sem_a, sem_b):
    cid = lax.axis_index("c")
    sid = lax.axis_index("s")
    wid = sid * _NC + cid
    base = wid * _TPW
    pltpu.sync_copy(x_hbm.at[pl.ds(base, _TPW)], xv)
    lanes = lax.iota(jnp.int32, 16)

    def build(c, carry):
        crow = jnp.full((16,), c, dtype=jnp.int32)
        for k in range(_CT // 16):
            pos = c * _CT + k * 16 + lanes
            v = plsc.load_gather(xv, [pos])
            cols = k * 16 + lanes
            plsc.store_scatter(idxb, [crow, cols], v)
            for e in range(_E):
                plsc.store_scatter(idx8, [crow + e * _NCH, cols],
                                   v + e * _VOCAB)
        return carry

    lax.fori_loop(0, _NCH, build, 0)

    def step(c, carry):
        t0 = base + c * _CT
        cp_b = pltpu.async_copy(w4_hbm.at[idxb.at[c]], buf_b, sem_b)
        for e in range(_E):
            cp_a = pltpu.async_copy(t8_hbm.at[idx8.at[e * _NCH + c]],
                                    buf_a, sem_a)
            cp_a.wait()
            pltpu.sync_copy(buf_a, out_hbm.at[pl.ds(t0, _CT), e])
        cp_b.wait()
        pltpu.sync_copy(buf_b, out_hbm.at[pl.ds(t0, _CT), pl.ds(_E, 4)])
        return carry

    lax.fori_loop(0, _NCH, step, 0)


@jax.jit
def _lookup(x_flat, t8, w4):
    f = functools.partial(
        pl.kernel,
        mesh=plsc.VectorSubcoreMesh(core_axis_name="c", subcore_axis_name="s"),
        out_type=jax.ShapeDtypeStruct((_N, _E + 4, _D_SUB), jnp.float32),
        scratch_types=[
            pltpu.VMEM((_TPW,), jnp.int32),
            pltpu.VMEM((_E * _NCH, _CT), jnp.int32),
            pltpu.VMEM((_NCH, _CT), jnp.int32),
            pltpu.VMEM((_CT, _D_SUB), jnp.float32),
            pltpu.VMEM((_CT, 4, _D_SUB), jnp.float32),
            pltpu.SemaphoreType.DMA,
            pltpu.SemaphoreType.DMA,
        ],
    )(_body)
    return f(x_flat, t8, w4)


def kernel(x, tables, W_train):
    x_flat = x.reshape(_N).astype(jnp.int32)
    t8 = tables.reshape(_E * _VOCAB, _D_SUB)
    w4 = W_train.reshape(_VOCAB, 4, _D_SUB)
    out = _lookup(x_flat, t8, w4)
    return out.reshape(_B, _L, _E * _D_SUB + _D_TRAIN)


# speedup vs baseline: 1.9457x; 1.9457x over previous
import functools

import jax
import jax.numpy as jnp
from jax import lax
from jax.experimental import pallas as pl
from jax.experimental.pallas import tpu as pltpu
from jax.experimental.pallas import tpu_sc as plsc

_VOCAB = 100000
_E = 8
_D_SUB = 128
_D_TRAIN = 512
_B, _L = 1024, 50
_N = _B * _L
_NC = 2
_NS = 16
_NW = _NC * _NS
_TPW = _N // _NW
_CT = 80
_NCH = _TPW // _CT


def _body(x_hbm, t8_hbm, w4_hbm, out_hbm, xv, idx8, idxb, buf_a, buf_b,
          sem_a, sem_b):
    cid = lax.axis_index("c")
    sid = lax.axis_index("s")
    wid = sid * _NC + cid
    base = wid * _TPW
    pltpu.sync_copy(x_hbm.at[pl.ds(base, _TPW)], xv)

    def build(c, carry):
        for k in range(_CT // 16):
            v = xv[pl.ds(c * _CT + k * 16, 16)]
            idxb[c, pl.ds(k * 16, 16)] = v
            for e in range(_E):
                idx8[e * _NCH + c, pl.ds(k * 16, 16)] = v + e * _VOCAB
        return carry

    lax.fori_loop(0, _NCH, build, 0)

    def step(c, carry):
        t0 = base + c * _CT
        cp_b = pltpu.async_copy(w4_hbm.at[idxb.at[c]], buf_b, sem_b)
        for e in range(_E):
            cp_a = pltpu.async_copy(t8_hbm.at[idx8.at[e * _NCH + c]],
                                    buf_a, sem_a)
            cp_a.wait()
            pltpu.sync_copy(buf_a, out_hbm.at[pl.ds(t0, _CT), e])
        cp_b.wait()
        pltpu.sync_copy(buf_b, out_hbm.at[pl.ds(t0, _CT), pl.ds(_E, 4)])
        return carry

    lax.fori_loop(0, _NCH, step, 0)


@jax.jit
def _lookup(x_flat, t8, w4):
    f = functools.partial(
        pl.kernel,
        mesh=plsc.VectorSubcoreMesh(core_axis_name="c", subcore_axis_name="s"),
        out_type=jax.ShapeDtypeStruct((_N, _E + 4, _D_SUB), jnp.float32),
        scratch_types=[
            pltpu.VMEM((_TPW,), jnp.int32),
            pltpu.VMEM((_E * _NCH, _CT), jnp.int32),
            pltpu.VMEM((_NCH, _CT), jnp.int32),
            pltpu.VMEM((_CT, _D_SUB), jnp.float32),
            pltpu.VMEM((_CT, 4, _D_SUB), jnp.float32),
            pltpu.SemaphoreType.DMA,
            pltpu.SemaphoreType.DMA,
        ],
    )(_body)
    return f(x_flat, t8, w4)


def kernel(x, tables, W_train):
    x_flat = x.reshape(_N).astype(jnp.int32)
    t8 = tables.reshape(_E * _VOCAB, _D_SUB)
    w4 = W_train.reshape(_VOCAB, 4, _D_SUB)
    out = _lookup(x_flat, t8, w4)
    return out.reshape(_B, _L, _E * _D_SUB + _D_TRAIN)


# speedup vs baseline: 2.1471x; 1.1035x over previous
import functools

import jax
import jax.numpy as jnp
from jax import lax
from jax.experimental import pallas as pl
from jax.experimental.pallas import tpu as pltpu
from jax.experimental.pallas import tpu_sc as plsc

_VOCAB = 100000
_E = 8
_D_SUB = 128
_D_TRAIN = 512
_B, _L = 1024, 50
_N = _B * _L
_NC = 2
_NS = 16
_NW = _NC * _NS
_TPW = _N // _NW
_CT = 80
_NCH = _TPW // _CT


def _body(x_hbm, t8_hbm, w4_hbm, out_hbm, xv, idx8, idxb,
          buf_a0, buf_a1, buf_b0, buf_b1,
          sga0, sga1, swa0, swa1, sgb0, sgb1, swb0, swb1):
    cid = lax.axis_index("c")
    sid = lax.axis_index("s")
    wid = sid * _NC + cid
    base = wid * _TPW
    buf_a = (buf_a0, buf_a1)
    buf_b = (buf_b0, buf_b1)
    sga = (sga0, sga1)
    swa = (swa0, swa1)
    sgb = (sgb0, sgb1)
    swb = (swb0, swb1)

    pltpu.sync_copy(x_hbm.at[pl.ds(base, _TPW)], xv)

    def build(c, carry):
        for k in range(_CT // 16):
            v = xv[pl.ds(c * _CT + k * 16, 16)]
            idxb[c, pl.ds(k * 16, 16)] = v
            for e in range(_E):
                idx8[e * _NCH + c, pl.ds(k * 16, 16)] = v + e * _VOCAB
        return carry

    lax.fori_loop(0, _NCH, build, 0)

    def b_gather(c, h):
        return pltpu.async_copy(w4_hbm.at[idxb.at[c]], buf_b[h], sgb[h])

    def b_write(c, h):
        return pltpu.make_async_copy(
            buf_b[h], out_hbm.at[pl.ds(base + c * _CT, _CT), pl.ds(_E, 4)],
            swb[h])

    def a_gather(c, e):
        return pltpu.async_copy(
            t8_hbm.at[idx8.at[e * _NCH + c]], buf_a[e % 2], sga[e % 2])

    def a_write(c, e):
        return pltpu.make_async_copy(
            buf_a[e % 2], out_hbm.at[pl.ds(base + c * _CT, _CT), e],
            swa[e % 2])

    def a_gather_wait(c, e):
        pltpu.make_async_copy(
            t8_hbm.at[idx8.at[e * _NCH + c]], buf_a[e % 2],
            sga[e % 2]).wait()

    b_gather(0, 0)
    b_gather(1, 1)

    def chunk(c, c2, h, first):
        pltpu.make_async_copy(w4_hbm.at[idxb.at[c]], buf_b[h], sgb[h]).wait()
        b_write(c, h).start()
        for e in range(_E):
            if e >= 2:
                a_write(c, e - 2).wait()
            elif not first:
                a_write(c - 1, 6 + e).wait()
            a_gather(c, e)
            if e >= 1:
                a_gather_wait(c, e - 1)
                a_write(c, e - 1).start()
            elif not first:
                a_gather_wait(c - 1, 7)
                a_write(c - 1, 7).start()
        b_write(c, h).wait()
        if isinstance(c2, int):
            if c2 < _NCH // 2 - 1:
                b_gather(c + 2, h)
        else:
            @pl.when(c2 < _NCH // 2 - 1)
            def _():
                b_gather(c + 2, h)

    chunk(0, 0, 0, True)
    chunk(1, 0, 1, False)

    def step(c2, carry):
        chunk(2 * c2, c2, 0, False)
        chunk(2 * c2 + 1, c2, 1, False)
        return carry

    lax.fori_loop(1, _NCH // 2, step, 0)

    a_gather_wait(_NCH - 1, 7)
    a_write(_NCH - 1, 7).start()
    a_write(_NCH - 1, 6).wait()
    a_write(_NCH - 1, 7).wait()


@jax.jit
def _lookup(x_flat, t8, w4):
    f = functools.partial(
        pl.kernel,
        mesh=plsc.VectorSubcoreMesh(core_axis_name="c", subcore_axis_name="s"),
        out_type=jax.ShapeDtypeStruct((_N, _E + 4, _D_SUB), jnp.float32),
        scratch_types=[
            pltpu.VMEM((_TPW,), jnp.int32),
            pltpu.VMEM((_E * _NCH, _CT), jnp.int32),
            pltpu.VMEM((_NCH, _CT), jnp.int32),
            pltpu.VMEM((_CT, _D_SUB), jnp.float32),
            pltpu.VMEM((_CT, _D_SUB), jnp.float32),
            pltpu.VMEM((_CT, 4, _D_SUB), jnp.float32),
            pltpu.VMEM((_CT, 4, _D_SUB), jnp.float32),
        ] + [pltpu.SemaphoreType.DMA] * 8,
    )(_body)
    return f(x_flat, t8, w4)


def kernel(x, tables, W_train):
    x_flat = x.reshape(_N).astype(jnp.int32)
    t8 = tables.reshape(_E * _VOCAB, _D_SUB)
    w4 = W_train.reshape(_VOCAB, 4, _D_SUB)
    out = _lookup(x_flat, t8, w4)
    return out.reshape(_B, _L, _E * _D_SUB + _D_TRAIN)


# speedup vs baseline: 2.9636x; 1.3803x over previous
import functools

import jax
import jax.numpy as jnp
from jax import lax
from jax.experimental import pallas as pl
from jax.experimental.pallas import tpu as pltpu
from jax.experimental.pallas import tpu_sc as plsc

_VOCAB = 100000
_E = 8
_D_SUB = 128
_D_TRAIN = 512
_B, _L = 1024, 50
_N = _B * _L
_NC = 2
_NS = 16
_NW = _NC * _NS
_TPW = _N // _NW
_CT = 80
_NCH = _TPW // _CT


def _body(x_hbm, t_hbm, w_hbm, out_hbm, xv, idxb,
          buf_a0, buf_a1, buf_b0, buf_b1,
          sga0, sga1, swa0, swa1, sgb0, sgb1, swb0, swb1):
    cid = lax.axis_index("c")
    sid = lax.axis_index("s")
    wid = sid * _NC + cid
    base = wid * _TPW
    buf_a = (buf_a0, buf_a1)
    buf_b = (buf_b0, buf_b1)
    sga = (sga0, sga1)
    swa = (swa0, swa1)
    sgb = (sgb0, sgb1)
    swb = (swb0, swb1)

    pltpu.sync_copy(x_hbm.at[pl.ds(base, _TPW)], xv)

    def build(c, carry):
        for k in range(_CT // 16):
            v = xv[pl.ds(c * _CT + k * 16, 16)]
            idxb[c, pl.ds(k * 16, 16)] = v
        return carry

    lax.fori_loop(0, _NCH, build, 0)

    def b_gather(c, h):
        return pltpu.async_copy(w_hbm.at[idxb.at[c]], buf_b[h], sgb[h])

    def b_write(c, h):
        return pltpu.make_async_copy(
            buf_b[h],
            out_hbm.at[pl.ds(base + c * _CT, _CT),
                       pl.ds(_E * _D_SUB, _D_TRAIN)],
            swb[h])

    def a_gather(c, e):
        return pltpu.async_copy(
            t_hbm.at[e].at[idxb.at[c]], buf_a[e % 2], sga[e % 2])

    def a_gather_wait(c, e):
        pltpu.make_async_copy(
            t_hbm.at[e].at[idxb.at[c]], buf_a[e % 2], sga[e % 2]).wait()

    def a_write(c, e):
        return pltpu.make_async_copy(
            buf_a[e % 2],
            out_hbm.at[pl.ds(base + c * _CT, _CT),
                       pl.ds(e * _D_SUB, _D_SUB)],
            swa[e % 2])

    b_gather(0, 0)
    b_gather(1, 1)

    def chunk(c, c2, h, first):
        pltpu.make_async_copy(w_hbm.at[idxb.at[c]], buf_b[h], sgb[h]).wait()
        b_write(c, h).start()
        for e in range(_E):
            if e >= 2:
                a_write(c, e - 2).wait()
            elif not first:
                a_write(c - 1, 6 + e).wait()
            a_gather(c, e)
            if e >= 1:
                a_gather_wait(c, e - 1)
                a_write(c, e - 1).start()
            elif not first:
                a_gather_wait(c - 1, 7)
                a_write(c - 1, 7).start()
        b_write(c, h).wait()
        if isinstance(c2, int):
            if c2 < _NCH // 2 - 1:
                b_gather(c + 2, h)
        else:
            @pl.when(c2 < _NCH // 2 - 1)
            def _():
                b_gather(c + 2, h)

    chunk(0, 0, 0, True)
    chunk(1, 0, 1, False)

    def step(c2, carry):
        chunk(2 * c2, c2, 0, False)
        chunk(2 * c2 + 1, c2, 1, False)
        return carry

    lax.fori_loop(1, _NCH // 2, step, 0)

    a_gather_wait(_NCH - 1, 7)
    a_write(_NCH - 1, 7).start()
    a_write(_NCH - 1, 6).wait()
    a_write(_NCH - 1, 7).wait()


@jax.jit
def _lookup(x_flat, tables, W_train):
    f = functools.partial(
        pl.kernel,
        mesh=plsc.VectorSubcoreMesh(core_axis_name="c", subcore_axis_name="s"),
        out_type=jax.ShapeDtypeStruct((_N, _E * _D_SUB + _D_TRAIN),
                                      jnp.float32),
        scratch_types=[
            pltpu.VMEM((_TPW,), jnp.int32),
            pltpu.VMEM((_NCH, _CT), jnp.int32),
            pltpu.VMEM((_CT, _D_SUB), jnp.float32),
            pltpu.VMEM((_CT, _D_SUB), jnp.float32),
            pltpu.VMEM((_CT, _D_TRAIN), jnp.float32),
            pltpu.VMEM((_CT, _D_TRAIN), jnp.float32),
        ] + [pltpu.SemaphoreType.DMA] * 8,
    )(_body)
    return f(x_flat, tables, W_train)


def kernel(x, tables, W_train):
    x_flat = x.reshape(_N).astype(jnp.int32)
    out = _lookup(x_flat, tables, W_train)
    return out.reshape(_B, _L, _E * _D_SUB + _D_TRAIN)


# speedup vs baseline: 9.0496x; 3.0536x over previous
import functools

import jax
import jax.numpy as jnp
from jax import lax
from jax.experimental import pallas as pl
from jax.experimental.pallas import tpu as pltpu
from jax.experimental.pallas import tpu_sc as plsc

_VOCAB = 100000
_E = 8
_D_SUB = 128
_D_TRAIN = 512
_B, _L = 1024, 50
_N = _B * _L
_NC = 2
_NS = 16
_NW = _NC * _NS
_TPW = _N // _NW
_CT = 80
_NCH = _TPW // _CT


def _body(x_hbm, t_hbm, w_hbm, out_hbm, xv, idxb,
          buf_a0, buf_a1, buf_b0, buf_b1,
          sga0, sga1, swa0, swa1, sgb0, sgb1, swb0, swb1):
    cid = lax.axis_index("c")
    sid = lax.axis_index("s")
    wid = sid * _NC + cid
    base = wid * _TPW
    buf_a = (buf_a0, buf_a1)
    buf_b = (buf_b0, buf_b1)
    sga = (sga0, sga1)
    swa = (swa0, swa1)
    sgb = (sgb0, sgb1)
    swb = (swb0, swb1)

    pltpu.sync_copy(x_hbm.at[pl.ds(base, _TPW)], xv)

    def build(c, carry):
        for k in range(_CT // 16):
            v = xv[pl.ds(c * _CT + k * 16, 16)]
            idxb[c, pl.ds(k * 16, 16)] = v
        return carry

    lax.fori_loop(0, _NCH, build, 0)

    def b_gather(c, h):
        return pltpu.async_copy(w_hbm.at[idxb.at[c]], buf_b[h], sgb[h])

    def b_write(c, h):
        return pltpu.make_async_copy(
            buf_b[h],
            out_hbm.at[pl.ds(base + c * _CT, _CT),
                       pl.ds(_E * _D_SUB, _D_TRAIN)],
            swb[h])

    def a_gather(c, e):
        return pltpu.async_copy(
            t_hbm.at[e].at[idxb.at[c]], buf_a[e % 2], sga[e % 2])

    def a_gather_wait(c, e):
        pltpu.make_async_copy(
            t_hbm.at[e].at[idxb.at[c]], buf_a[e % 2], sga[e % 2]).wait()

    def a_write(c, e):
        return pltpu.make_async_copy(
            buf_a[e % 2],
            out_hbm.at[pl.ds(base + c * _CT, _CT),
                       pl.ds(e * _D_SUB, _D_SUB)],
            swa[e % 2])

    b_gather(0, 0)
    b_gather(1, 1)

    def chunk(c, c2, h, first):
        pltpu.make_async_copy(w_hbm.at[idxb.at[c]], buf_b[h], sgb[h]).wait()
        b_write(c, h).start()
        for e in range(_E):
            if e >= 2:
                a_write(c, e - 2).wait()
            elif not first:
                a_write(c - 1, 6 + e).wait()
            a_gather(c, e)
            if e >= 1:
                a_gather_wait(c, e - 1)
                a_write(c, e - 1).start()
            elif not first:
                a_gather_wait(c - 1, 7)
                a_write(c - 1, 7).start()
        b_write(c, h).wait()
        if isinstance(c2, int):
            if c2 < _NCH // 2 - 1:
                b_gather(c + 2, h)
        else:
            @pl.when(c2 < _NCH // 2 - 1)
            def _():
                b_gather(c + 2, h)

    chunk(0, 0, 0, True)
    chunk(1, 0, 1, False)

    def step(c2, carry):
        chunk(2 * c2, c2, 0, False)
        chunk(2 * c2 + 1, c2, 1, False)
        return carry

    lax.fori_loop(1, _NCH // 2, step, 0)

    a_gather_wait(_NCH - 1, 7)
    a_write(_NCH - 1, 7).start()
    a_write(_NCH - 1, 6).wait()
    a_write(_NCH - 1, 7).wait()


@jax.jit
def _lookup(x_flat, tables, W_train):
    f = functools.partial(
        pl.kernel,
        mesh=plsc.VectorSubcoreMesh(core_axis_name="c", subcore_axis_name="s"),
        out_type=jax.ShapeDtypeStruct((_N, _E * _D_SUB + _D_TRAIN),
                                      jnp.float32),
        scratch_types=[
            pltpu.VMEM((_TPW,), jnp.int32),
            pltpu.VMEM((_NCH, _CT), jnp.int32),
            pltpu.VMEM((_CT, _D_SUB), jnp.float32),
            pltpu.VMEM((_CT, _D_SUB), jnp.float32),
            pltpu.VMEM((_CT, _D_TRAIN), jnp.float32),
            pltpu.VMEM((_CT, _D_TRAIN), jnp.float32),
        ] + [pltpu.SemaphoreType.DMA] * 8,
    )(_body)
    return f(x_flat, tables, W_train)


def kernel(x, tables, W_train):
    x_lm = x.T.reshape(_N).astype(jnp.int32)
    out = _lookup(x_lm, tables, W_train)
    out = out.reshape(_L, _B, _E * _D_SUB + _D_TRAIN)
    return out.transpose(1, 0, 2)


# speedup vs baseline: 9.2705x; 1.0244x over previous
import functools

import jax
import jax.numpy as jnp
from jax import lax
from jax.experimental import pallas as pl
from jax.experimental.pallas import tpu as pltpu
from jax.experimental.pallas import tpu_sc as plsc

_VOCAB = 100000
_E = 8
_D_SUB = 128
_D_TRAIN = 512
_B, _L = 1024, 50
_N = _B * _L
_NC = 2
_NS = 16
_NW = _NC * _NS
_TPW = _N // _NW
_CT = 80
_NCH = _TPW // _CT


def _body(x_hbm, t_hbm, w_hbm, out_hbm, xv, idxb,
          buf_a0, buf_a1, buf_a2, buf_a3, buf_b0, buf_b1,
          sga0, sga1, sga2, sga3, swa0, swa1, swa2, swa3,
          sgb0, sgb1, swb0, swb1):
    cid = lax.axis_index("c")
    sid = lax.axis_index("s")
    wid = sid * _NC + cid
    base = wid * _TPW
    buf_a = (buf_a0, buf_a1, buf_a2, buf_a3)
    buf_b = (buf_b0, buf_b1)
    sga = (sga0, sga1, sga2, sga3)
    swa = (swa0, swa1, swa2, swa3)
    sgb = (sgb0, sgb1)
    swb = (swb0, swb1)

    pltpu.sync_copy(x_hbm.at[pl.ds(base, _TPW)], xv)

    def build(c, carry):
        for k in range(_CT // 16):
            v = xv[pl.ds(c * _CT + k * 16, 16)]
            idxb[c, pl.ds(k * 16, 16)] = v
        return carry

    lax.fori_loop(0, _NCH, build, 0)

    def b_gather(c, h):
        return pltpu.async_copy(w_hbm.at[idxb.at[c]], buf_b[h], sgb[h])

    def b_write(c, h):
        return pltpu.make_async_copy(
            buf_b[h],
            out_hbm.at[pl.ds(base + c * _CT, _CT),
                       pl.ds(_E * _D_SUB, _D_TRAIN)],
            swb[h])

    def a_gather(c, e):
        return pltpu.async_copy(
            t_hbm.at[e].at[idxb.at[c]], buf_a[e % 4], sga[e % 4])

    def a_gather_wait(c, e):
        pltpu.make_async_copy(
            t_hbm.at[e].at[idxb.at[c]], buf_a[e % 4], sga[e % 4]).wait()

    def a_write(c, e):
        return pltpu.make_async_copy(
            buf_a[e % 4],
            out_hbm.at[pl.ds(base + c * _CT, _CT),
                       pl.ds(e * _D_SUB, _D_SUB)],
            swa[e % 4])

    b_gather(0, 0)
    b_gather(1, 1)

    def chunk(c, c2, h, first):
        pltpu.make_async_copy(w_hbm.at[idxb.at[c]], buf_b[h], sgb[h]).wait()
        b_write(c, h).start()
        for e in range(_E):
            if e >= 4:
                a_write(c, e - 4).wait()
            elif not first:
                a_write(c - 1, 4 + e).wait()
            a_gather(c, e)
            if e >= 1:
                a_gather_wait(c, e - 1)
                a_write(c, e - 1).start()
            elif not first:
                a_gather_wait(c - 1, 7)
                a_write(c - 1, 7).start()
        b_write(c, h).wait()
        if isinstance(c2, int):
            if c2 < _NCH // 2 - 1:
                b_gather(c + 2, h)
        else:
            @pl.when(c2 < _NCH // 2 - 1)
            def _():
                b_gather(c + 2, h)

    chunk(0, 0, 0, True)
    chunk(1, 0, 1, False)

    def step(c2, carry):
        chunk(2 * c2, c2, 0, False)
        chunk(2 * c2 + 1, c2, 1, False)
        return carry

    lax.fori_loop(1, _NCH // 2, step, 0)

    a_gather_wait(_NCH - 1, 7)
    a_write(_NCH - 1, 7).start()
    for e in (4, 5, 6, 7):
        a_write(_NCH - 1, e).wait()


@jax.jit
def _lookup(x_flat, tables, W_train):
    f = functools.partial(
        pl.kernel,
        mesh=plsc.VectorSubcoreMesh(core_axis_name="c", subcore_axis_name="s"),
        out_type=jax.ShapeDtypeStruct((_N, _E * _D_SUB + _D_TRAIN),
                                      jnp.float32),
        scratch_types=[
            pltpu.VMEM((_TPW,), jnp.int32),
            pltpu.VMEM((_NCH, _CT), jnp.int32),
            pltpu.VMEM((_CT, _D_SUB), jnp.float32),
            pltpu.VMEM((_CT, _D_SUB), jnp.float32),
            pltpu.VMEM((_CT, _D_SUB), jnp.float32),
            pltpu.VMEM((_CT, _D_SUB), jnp.float32),
            pltpu.VMEM((_CT, _D_TRAIN), jnp.float32),
            pltpu.VMEM((_CT, _D_TRAIN), jnp.float32),
        ] + [pltpu.SemaphoreType.DMA] * 12,
    )(_body)
    return f(x_flat, tables, W_train)


def kernel(x, tables, W_train):
    x_lm = x.T.reshape(_N).astype(jnp.int32)
    out = _lookup(x_lm, tables, W_train)
    out = out.reshape(_L, _B, _E * _D_SUB + _D_TRAIN)
    return out.transpose(1, 0, 2)
